# baseline (speedup 1.0000x reference)
import functools

import jax
import jax.numpy as jnp
from jax import lax
from jax.experimental import pallas as pl
from jax.experimental.pallas import tpu as pltpu
from jax.experimental.pallas import tpu_sc as plsc

NUM_TIME = 288
TIME_SIZE = 64
DAY_SIZE = 32
NUM_DAY = 7
OUT_SIZE = TIME_SIZE + DAY_SIZE
NUM_FUSED = NUM_TIME * NUM_DAY
NUM_HOT = NUM_DAY * NUM_DAY
NUM_HOT_PAD = 56
FPAD = 128

NC = 2
NS = 16
NW = NC * NS
LANES = 16

CHUNK = 256
GROUPS = CHUNK // LANES
GROW = 128
NGATHER = CHUNK // GROW


def _sc_embed(n_tokens):
  per_w = n_tokens // NW
  iters = per_w // CHUNK
  assert per_w % CHUNK == 0 and iters % 2 == 0

  mesh = plsc.VectorSubcoreMesh(core_axis_name="c", subcore_axis_name="s")

  @functools.partial(
      pl.kernel,
      mesh=mesh,
      out_type=jax.ShapeDtypeStruct((n_tokens, FPAD), jnp.float32),
      compiler_params=pltpu.CompilerParams(
          needs_layout_passes=False, use_tc_tiling_on_sc=True
      ),
      scratch_types=[
          pltpu.VMEM((2 * CHUNK,), jnp.int32),
          pltpu.VMEM((2 * CHUNK,), jnp.int32),
          pltpu.VMEM((2, NGATHER, GROW), jnp.int32),
          pltpu.VMEM((2, CHUNK), jnp.int32),
          pltpu.VMEM((NUM_HOT_PAD, FPAD), jnp.float32),
          pltpu.VMEM((CHUNK, FPAD), jnp.float32),
          pltpu.VMEM((CHUNK, FPAD), jnp.float32),
          pltpu.SMEM((2,), jnp.int32),
          pltpu.SemaphoreType.DMA,
          pltpu.SemaphoreType.DMA,
          pltpu.SemaphoreType.DMA,
          pltpu.SemaphoreType.DMA,
          pltpu.SemaphoreType.DMA,
          pltpu.SemaphoreType.DMA,
      ],
  )
  def k(dt_hbm, fused_hbm, out_hbm,
        dt0, dt1, p_v, pf_v, tab_v, blk0, blk1, mx_s,
        is0, is1, gs0, gs1, ws0, ws1):
    dt_v = (dt0, dt1)
    wid = lax.axis_index("s") * NC + lax.axis_index("c")
    base = wid * per_w
    blk = (blk0, blk1)
    isem = (is0, is1)
    gsem = (gs0, gs1)
    wsem = (ws0, ws1)

    def tok0(i):
      return base + i * CHUNK

    def fire_idx(i, b):
      pltpu.async_copy(
          dt_hbm.at[pl.ds(2 * tok0(i), 2 * CHUNK)], dt_v[b], isem[b])

    def wait_idx(i, b):
      pltpu.make_async_copy(
          dt_hbm.at[pl.ds(2 * tok0(i), 2 * CHUNK)], dt_v[b], isem[b]).wait()

    def compute_p(b):
      gpr = GROW // LANES
      mx = jnp.zeros((LANES,), jnp.int32)
      iota2 = lax.iota(jnp.int32, LANES) * 2
      for g in range(GROUPS):
        ii = iota2 + 2 * g * LANES
        d = plsc.load_gather(dt_v[b], [ii])
        t = plsc.load_gather(dt_v[b], [ii + 1])
        mx = jnp.maximum(mx, jnp.maximum(d, t))
        d = jnp.clip(d, 0, NUM_DAY - 1)
        t = jnp.clip(t, 0, NUM_TIME - 1)
        p = t * NUM_DAY + d
        p_v[b, g // gpr, pl.ds((g % gpr) * LANES, LANES)] = p
        pf_v[b, pl.ds(g * LANES, LANES)] = p
      mx_s[b] = lax.reduce_max(mx, (0,))

    iota = lax.iota(jnp.int32, LANES)

    def build_fast(b):
      half = OUT_SIZE // 2
      @pl.loop(0, GROUPS)
      def _(g):
        pvec = pf_v[b, pl.ds(g * LANES, LANES)]
        tokvec = g * LANES + iota
        col_a = iota
        col_b = iota + half
        for k in range(half):
          va = plsc.load_gather(tab_v, [pvec, col_a])
          vb = plsc.load_gather(tab_v, [pvec, col_b])
          plsc.store_scatter(blk[b], [tokvec, col_a], va)
          plsc.store_scatter(blk[b], [tokvec, col_b], vb)
          if k + 1 < half:
            col_a = col_a + 1
            col_a = jnp.where(col_a >= OUT_SIZE, col_a - OUT_SIZE, col_a)
            col_b = col_b + 1
            col_b = jnp.where(col_b >= OUT_SIZE, col_b - OUT_SIZE, col_b)

    def fire_gathers(b):
      for j in range(NGATHER):
        pltpu.async_copy(
            fused_hbm.at[p_v.at[b, j]],
            blk[b].at[pl.ds(j * GROW, GROW)], gsem[b])

    def wait_gathers(b):
      for j in range(NGATHER):
        pltpu.make_async_copy(
            fused_hbm.at[p_v.at[b, j]],
            blk[b].at[pl.ds(j * GROW, GROW)], gsem[b]).wait()

    def fire_wb(i, b):
      pltpu.async_copy(blk[b], out_hbm.at[pl.ds(tok0(i), CHUNK)], wsem[b])

    def wait_wb(i, b):
      pltpu.make_async_copy(
          blk[b], out_hbm.at[pl.ds(tok0(i), CHUNK)], wsem[b]).wait()

    pltpu.sync_copy(fused_hbm.at[pl.ds(0, NUM_HOT_PAD)], tab_v)

    fire_idx(0, 0)
    wait_idx(0, 0)
    compute_p(0)
    fire_idx(1, 1)

    @pl.loop(0, iters, step=2)
    def _(o):
      for b in (0, 1):
        i = o + b
        nb = 1 - b

        @pl.when(i > 1)
        def _():
          wait_wb(i - 2, b)

        hot = mx_s[b] < NUM_DAY

        @pl.when(hot)
        def _():
          build_fast(b)

        @pl.when(jnp.logical_not(hot))
        def _():
          fire_gathers(b)
          wait_gathers(b)

        fire_wb(i, b)

        @pl.when(i + 1 < iters)
        def _():
          wait_idx(i + 1, nb)
          compute_p(nb)

          @pl.when(i + 2 < iters)
          def _():
            fire_idx(i + 2, b)

    wait_wb(iters - 2, 0)
    wait_wb(iters - 1, 1)

  return k


def kernel(data_cat, table_day, table_time):
  B, T, _ = data_cat.shape
  n = B * T
  dt_flat = data_cat.astype(jnp.int32).reshape(2 * n)
  f_time = jnp.broadcast_to(
      table_time[:, None, :], (NUM_TIME, NUM_DAY, TIME_SIZE)
  ).reshape(NUM_FUSED, TIME_SIZE)
  f_day = jnp.broadcast_to(
      table_day[None, :, :], (NUM_TIME, NUM_DAY, DAY_SIZE)
  ).reshape(NUM_FUSED, DAY_SIZE)
  f_pad = jnp.zeros((NUM_FUSED, FPAD - OUT_SIZE), jnp.float32)
  fused = jnp.concatenate([f_time, f_day, f_pad], axis=1)
  out = _sc_embed(n)(dt_flat, fused)
  return out[:, :OUT_SIZE].reshape(B, T, OUT_SIZE)


# speedup vs baseline: 2.6617x; 2.6617x over previous
import functools

import jax
import jax.numpy as jnp
from jax import lax
from jax.experimental import pallas as pl
from jax.experimental.pallas import tpu as pltpu
from jax.experimental.pallas import tpu_sc as plsc

NUM_TIME = 288
TIME_SIZE = 64
DAY_SIZE = 32
NUM_DAY = 7
OUT_SIZE = TIME_SIZE + DAY_SIZE
NUM_FUSED = NUM_TIME * NUM_DAY
NUM_HOT = NUM_DAY * NUM_DAY
NUM_HOT_PAD = 56
FPAD = 128

NC = 2
NS = 16
NW = NC * NS
LANES = 16

CHUNK = 256
GROUPS = CHUNK // LANES
GROW = 128
NGATHER = CHUNK // GROW


def _sc_embed(n_tokens):
  per_w = n_tokens // NW
  iters = per_w // CHUNK
  assert per_w % CHUNK == 0 and iters % 2 == 0

  mesh = plsc.VectorSubcoreMesh(core_axis_name="c", subcore_axis_name="s")

  @functools.partial(
      pl.kernel,
      mesh=mesh,
      out_type=jax.ShapeDtypeStruct((n_tokens, FPAD), jnp.float32),
      compiler_params=pltpu.CompilerParams(
          needs_layout_passes=False, use_tc_tiling_on_sc=True
      ),
      scratch_types=[
          pltpu.VMEM((CHUNK,), jnp.int32),
          pltpu.VMEM((CHUNK,), jnp.int32),
          pltpu.VMEM((CHUNK,), jnp.int32),
          pltpu.VMEM((CHUNK,), jnp.int32),
          pltpu.VMEM((2, NGATHER, GROW), jnp.int32),
          pltpu.VMEM((2, CHUNK), jnp.int32),
          pltpu.VMEM((NUM_HOT_PAD, FPAD), jnp.float32),
          pltpu.VMEM((CHUNK, FPAD), jnp.float32),
          pltpu.VMEM((CHUNK, FPAD), jnp.float32),
          pltpu.SMEM((2,), jnp.int32),
          pltpu.SemaphoreType.DMA,
          pltpu.SemaphoreType.DMA,
          pltpu.SemaphoreType.DMA,
          pltpu.SemaphoreType.DMA,
          pltpu.SemaphoreType.DMA,
          pltpu.SemaphoreType.DMA,
      ],
  )
  def k(day_hbm, time_hbm, fused_hbm, out_hbm,
        d0, d1, t0, t1, p_v, pf_v, tab_v, blk0, blk1, mx_s,
        is0, is1, gs0, gs1, ws0, ws1):
    d_v = (d0, d1)
    t_v = (t0, t1)
    wid = lax.axis_index("s") * NC + lax.axis_index("c")
    base = wid * per_w
    blk = (blk0, blk1)
    isem = (is0, is1)
    gsem = (gs0, gs1)
    wsem = (ws0, ws1)

    def tok0(i):
      return base + i * CHUNK

    def fire_idx(i, b):
      pltpu.async_copy(day_hbm.at[pl.ds(tok0(i), CHUNK)], d_v[b], isem[b])
      pltpu.async_copy(time_hbm.at[pl.ds(tok0(i), CHUNK)], t_v[b], isem[b])

    def wait_idx(i, b):
      pltpu.make_async_copy(
          day_hbm.at[pl.ds(tok0(i), CHUNK)], d_v[b], isem[b]).wait()
      pltpu.make_async_copy(
          time_hbm.at[pl.ds(tok0(i), CHUNK)], t_v[b], isem[b]).wait()

    def compute_p(b):
      gpr = GROW // LANES
      mx = jnp.zeros((LANES,), jnp.int32)
      for g in range(GROUPS):
        d = d_v[b][pl.ds(g * LANES, LANES)]
        t = t_v[b][pl.ds(g * LANES, LANES)]
        mx = jnp.maximum(mx, jnp.maximum(d, t))
        d = jnp.clip(d, 0, NUM_DAY - 1)
        t = jnp.clip(t, 0, NUM_TIME - 1)
        p = t * NUM_DAY + d
        p_v[b, g // gpr, pl.ds((g % gpr) * LANES, LANES)] = p
        pf_v[b, pl.ds(g * LANES, LANES)] = p
      mx_s[b] = lax.reduce_max(mx, (0,))

    iota = lax.iota(jnp.int32, LANES)

    def build_fast(b):
      half = OUT_SIZE // 2
      @pl.loop(0, GROUPS)
      def _(g):
        pvec = pf_v[b, pl.ds(g * LANES, LANES)]
        tokvec = g * LANES + iota
        col_a = iota
        col_b = iota + half
        for k in range(half):
          va = plsc.load_gather(tab_v, [pvec, col_a])
          vb = plsc.load_gather(tab_v, [pvec, col_b])
          plsc.store_scatter(blk[b], [tokvec, col_a], va)
          plsc.store_scatter(blk[b], [tokvec, col_b], vb)
          if k + 1 < half:
            col_a = col_a + 1
            col_a = jnp.where(col_a >= OUT_SIZE, col_a - OUT_SIZE, col_a)
            col_b = col_b + 1
            col_b = jnp.where(col_b >= OUT_SIZE, col_b - OUT_SIZE, col_b)

    def fire_gathers(b):
      for j in range(NGATHER):
        pltpu.async_copy(
            fused_hbm.at[p_v.at[b, j]],
            blk[b].at[pl.ds(j * GROW, GROW)], gsem[b])

    def wait_gathers(b):
      for j in range(NGATHER):
        pltpu.make_async_copy(
            fused_hbm.at[p_v.at[b, j]],
            blk[b].at[pl.ds(j * GROW, GROW)], gsem[b]).wait()

    def fire_wb(i, b):
      pltpu.async_copy(blk[b], out_hbm.at[pl.ds(tok0(i), CHUNK)], wsem[b])

    def wait_wb(i, b):
      pltpu.make_async_copy(
          blk[b], out_hbm.at[pl.ds(tok0(i), CHUNK)], wsem[b]).wait()

    pltpu.sync_copy(fused_hbm.at[pl.ds(0, NUM_HOT_PAD)], tab_v)

    fire_idx(0, 0)
    wait_idx(0, 0)
    compute_p(0)
    fire_idx(1, 1)

    @pl.loop(0, iters, step=2)
    def _(o):
      for b in (0, 1):
        i = o + b
        nb = 1 - b

        @pl.when(i > 1)
        def _():
          wait_wb(i - 2, b)

        hot = mx_s[b] < NUM_DAY

        @pl.when(hot)
        def _():
          build_fast(b)

        @pl.when(jnp.logical_not(hot))
        def _():
          fire_gathers(b)
          wait_gathers(b)

        fire_wb(i, b)

        @pl.when(i + 1 < iters)
        def _():
          wait_idx(i + 1, nb)
          compute_p(nb)

          @pl.when(i + 2 < iters)
          def _():
            fire_idx(i + 2, b)

    wait_wb(iters - 2, 0)
    wait_wb(iters - 1, 1)

  return k


def kernel(data_cat, table_day, table_time):
  B, T, _ = data_cat.shape
  n = B * T
  data_cat = data_cat.astype(jnp.int32)
  day_idx = (data_cat[:, :, 0] ^ 1).reshape(n) ^ 1
  time_idx = (data_cat[:, :, 1] ^ 1).reshape(n) ^ 1
  f_time = jnp.broadcast_to(
      table_time[:, None, :], (NUM_TIME, NUM_DAY, TIME_SIZE)
  ).reshape(NUM_FUSED, TIME_SIZE)
  f_day = jnp.broadcast_to(
      table_day[None, :, :], (NUM_TIME, NUM_DAY, DAY_SIZE)
  ).reshape(NUM_FUSED, DAY_SIZE)
  f_pad = jnp.zeros((NUM_FUSED, FPAD - OUT_SIZE), jnp.float32)
  fused = jnp.concatenate([f_time, f_day, f_pad], axis=1)
  out = _sc_embed(n)(day_idx, time_idx, fused)
  return out[:, :OUT_SIZE].reshape(B, T, OUT_SIZE)
